# Initial kernel scaffold; baseline (speedup 1.0000x reference)
import jax
import jax.numpy as jnp
from jax.experimental import pallas as pl


def kernel(adj, mask, gold_edge_index, gold_edge_attr, ent_indices):
    raise NotImplementedError("write your pallas kernel here")


# speedup vs baseline: 4.3026x; 4.3026x over previous
import functools

import jax
import jax.numpy as jnp
from jax import lax
from jax.experimental import pallas as pl
from jax.experimental.pallas import tpu as pltpu
from jax.experimental.pallas import tpu_sc as plsc

_B, _C, _N, _E = 4, 16, 1024, 4096
_NSUB = 16
_NCORE = 2
_TWORDS = (_N * _N) // _NSUB
_TROWS = _N // _NSUB
_CHUNK = 16
_NCHUNKS = _E // _CHUNK
_BIG = 2 ** 30


def _sc_scatter_body(pos_hbm, attr_hbm, out_hbm, pos_v, attr_v, grid_v, key_v):
  cid = lax.axis_index("c")
  sid = lax.axis_index("s")
  lane = lax.iota(jnp.int32, _CHUNK)
  zeros16 = jnp.zeros((_CHUNK,), jnp.int32)

  for b in range(_B):
    @pl.when(cid == (b // 2))
    def _process():
      pltpu.sync_copy(pos_hbm.at[b], pos_v)
      pltpu.sync_copy(attr_hbm.at[b], attr_v)

      def _zero(i, _):
        base = i * 64
        grid_v[pl.ds(base, 16)] = zeros16
        grid_v[pl.ds(base + 16, 16)] = zeros16
        grid_v[pl.ds(base + 32, 16)] = zeros16
        grid_v[pl.ds(base + 48, 16)] = zeros16
        return 0
      lax.fori_loop(0, _TWORDS // 64, _zero, 0)

      def _chunk(i, _):
        pv = pos_v[pl.ds(i * _CHUNK, _CHUNK)]
        av = attr_v[pl.ds(i * _CHUNK, _CHUNK)]
        own = lax.shift_right_logical(pv, 16) == sid
        lp = jnp.bitwise_and(pv, jnp.int32(_TWORDS - 1))
        key = jnp.where(own, lp * 16 + lane, jnp.int32(_BIG))
        sk, sa = plsc.sort_key_val(key, av)
        key_v[pl.ds(0, 16)] = sk
        nxt = key_v[pl.ds(1, 16)]
        grp = lax.shift_right_logical(sk, 4)
        ngrp = lax.shift_right_logical(nxt, 4)
        keep = jnp.logical_and(
            jnp.logical_or(grp != ngrp, lane == 15), sk < _BIG)
        plsc.store_scatter(grid_v, [grp], sa, mask=keep)
        return 0
      lax.fori_loop(0, _NCHUNKS, _chunk, 0)

      pltpu.sync_copy(grid_v, out_hbm.at[b, pl.ds(sid * _TWORDS, _TWORDS)])


def _sc_scatter(pos, attr):
  mesh = plsc.VectorSubcoreMesh(core_axis_name="c", subcore_axis_name="s")
  return pl.kernel(
      _sc_scatter_body,
      out_type=jax.ShapeDtypeStruct((_B, _N * _N), jnp.int32),
      mesh=mesh,
      compiler_params=pltpu.CompilerParams(needs_layout_passes=False),
      scratch_types=[
          pltpu.VMEM((_E,), jnp.int32),
          pltpu.VMEM((_E,), jnp.int32),
          pltpu.VMEM((_TWORDS,), jnp.int32),
          pltpu.VMEM((32,), jnp.int32),
      ],
  )(pos, attr)


_TR = 128


def _tc_ce_body(adj_ref, ans_ref, mask_ref, sum_ref, cnt_ref):
  x = adj_ref[0]
  mx = jnp.max(x, axis=0)
  lse = jnp.log(jnp.sum(jnp.exp(x - mx[None]), axis=0)) + mx
  tgt = ans_ref[0]
  cidx = lax.broadcasted_iota(jnp.int32, x.shape, 0)
  sel = jnp.sum(jnp.where(cidx == tgt[None], x, 0.0), axis=0)
  mf = mask_ref[0].astype(jnp.float32)
  sum_ref[0, 0, 0] = jnp.sum((lse - sel) * mf)
  cnt_ref[0, 0, 0] = jnp.sum(mf)


def _tc_ce(adj, ans, mask):
  nr = _N // _TR
  sums, cnts = pl.pallas_call(
      _tc_ce_body,
      grid=(_B, nr),
      in_specs=[
          pl.BlockSpec((1, _C, _TR, _N), lambda b, r: (b, 0, r, 0)),
          pl.BlockSpec((1, _TR, _N), lambda b, r: (b, r, 0)),
          pl.BlockSpec((1, _TR, _N), lambda b, r: (b, r, 0)),
      ],
      out_specs=[
          pl.BlockSpec((1, 1, 1), lambda b, r: (b * nr + r, 0, 0),
                       memory_space=pltpu.SMEM),
          pl.BlockSpec((1, 1, 1), lambda b, r: (b * nr + r, 0, 0),
                       memory_space=pltpu.SMEM),
      ],
      out_shape=[
          jax.ShapeDtypeStruct((_B * nr, 1, 1), jnp.float32),
          jax.ShapeDtypeStruct((_B * nr, 1, 1), jnp.float32),
      ],
  )(adj, ans, mask)
  return sums.reshape(_B, nr), cnts.reshape(_B, nr)


@jax.jit
def kernel(adj, mask, gold_edge_index, gold_edge_attr, ent_indices):
  del ent_indices
  e = gold_edge_index.astype(jnp.int32)
  pos = e[:, :, 0] * _N + e[:, :, 1]
  attr = gold_edge_attr.astype(jnp.int32)
  ans = _sc_scatter(pos, attr)
  ans = ans.reshape(_B, _N, _N)
  sums, cnts = _tc_ce(adj, ans, mask)
  s = sums.sum(axis=1)
  c = cnts.sum(axis=1)
  return (s / jnp.maximum(c, 1.0)).sum() / _B


# speedup vs baseline: 4.9801x; 1.1575x over previous
import functools

import jax
import jax.numpy as jnp
from jax import lax
from jax.experimental import pallas as pl
from jax.experimental.pallas import tpu as pltpu
from jax.experimental.pallas import tpu_sc as plsc

_B, _C, _N, _E = 4, 16, 1024, 4096
_NSUB = 16
_NCORE = 2
_TWORDS = (_N * _N) // _NSUB
_TROWS = _N // _NSUB
_CHUNK = 16
_NCHUNKS = _E // _CHUNK
_BIG = 2 ** 30


def _sc_scatter_body(pos_hbm, attr_hbm, out_hbm, pos_v, attr_v, grid_v, key_v):
  cid = lax.axis_index("c")
  sid = lax.axis_index("s")
  lane = lax.iota(jnp.int32, _CHUNK)
  zeros16 = jnp.zeros((_CHUNK,), jnp.int32)

  for b in range(_B):
    @pl.when(cid == (b // 2))
    def _process():
      pltpu.sync_copy(pos_hbm.at[b], pos_v)
      pltpu.sync_copy(attr_hbm.at[b], attr_v)

      def _zero(i, _):
        base = i * 64
        grid_v[pl.ds(base, 16)] = zeros16
        grid_v[pl.ds(base + 16, 16)] = zeros16
        grid_v[pl.ds(base + 32, 16)] = zeros16
        grid_v[pl.ds(base + 48, 16)] = zeros16
        return 0
      lax.fori_loop(0, _TWORDS // 64, _zero, 0)

      def _chunk(i, _):
        pv = pos_v[pl.ds(i * _CHUNK, _CHUNK)]
        av = attr_v[pl.ds(i * _CHUNK, _CHUNK)]
        own = lax.shift_right_logical(pv, 16) == sid
        lp = jnp.bitwise_and(pv, jnp.int32(_TWORDS - 1))
        key = jnp.where(own, lp * 16 + lane, jnp.int32(_BIG))
        sk, sa = plsc.sort_key_val(key, av)
        key_v[pl.ds(0, 16)] = sk
        nxt = key_v[pl.ds(1, 16)]
        grp = lax.shift_right_logical(sk, 4)
        ngrp = lax.shift_right_logical(nxt, 4)
        keep = jnp.logical_and(
            jnp.logical_or(grp != ngrp, lane == 15), sk < _BIG)
        plsc.store_scatter(grid_v, [grp], sa, mask=keep)
        return 0
      lax.fori_loop(0, _NCHUNKS, _chunk, 0)

      pltpu.sync_copy(grid_v, out_hbm.at[b, pl.ds(sid * _TWORDS, _TWORDS)])


def _sc_scatter(pos, attr):
  mesh = plsc.VectorSubcoreMesh(core_axis_name="c", subcore_axis_name="s")
  return pl.kernel(
      _sc_scatter_body,
      out_type=jax.ShapeDtypeStruct((_B, _N * _N), jnp.int32),
      mesh=mesh,
      compiler_params=pltpu.CompilerParams(needs_layout_passes=False),
      scratch_types=[
          pltpu.VMEM((_E,), jnp.int32),
          pltpu.VMEM((_E,), jnp.int32),
          pltpu.VMEM((_TWORDS,), jnp.int32),
          pltpu.VMEM((32,), jnp.int32),
      ],
  )(pos, attr)


_TR = 128


_SR = 8


def _tc_ce_body(adj_ref, ans_ref, mask_ref, sum_ref, cnt_ref):
  total = jnp.float32(0.0)
  count = jnp.float32(0.0)
  for t in range(_TR // _SR):
    r0 = t * _SR
    m = adj_ref[0, 0, r0:r0 + _SR, :]
    for c in range(1, _C):
      m = jnp.maximum(m, adj_ref[0, c, r0:r0 + _SR, :])
    tgt = ans_ref[0, r0:r0 + _SR, :]
    s = jnp.zeros((_SR, _N), jnp.float32)
    sel = jnp.zeros((_SR, _N), jnp.float32)
    for c in range(_C):
      xc = adj_ref[0, c, r0:r0 + _SR, :]
      s = s + jnp.exp(xc - m)
      sel = sel + jnp.where(tgt == c, xc, 0.0)
    mf = mask_ref[0, r0:r0 + _SR, :].astype(jnp.float32)
    nll = (jnp.log(s) + m - sel) * mf
    total = total + jnp.sum(nll)
    count = count + jnp.sum(mf)
  sum_ref[0, 0, 0] = total
  cnt_ref[0, 0, 0] = count


def _tc_ce(adj, ans, mask):
  nr = _N // _TR
  sums, cnts = pl.pallas_call(
      _tc_ce_body,
      grid=(_B, nr),
      in_specs=[
          pl.BlockSpec((1, _C, _TR, _N), lambda b, r: (b, 0, r, 0)),
          pl.BlockSpec((1, _TR, _N), lambda b, r: (b, r, 0)),
          pl.BlockSpec((1, _TR, _N), lambda b, r: (b, r, 0)),
      ],
      out_specs=[
          pl.BlockSpec((1, 1, 1), lambda b, r: (b * nr + r, 0, 0),
                       memory_space=pltpu.SMEM),
          pl.BlockSpec((1, 1, 1), lambda b, r: (b * nr + r, 0, 0),
                       memory_space=pltpu.SMEM),
      ],
      out_shape=[
          jax.ShapeDtypeStruct((_B * nr, 1, 1), jnp.float32),
          jax.ShapeDtypeStruct((_B * nr, 1, 1), jnp.float32),
      ],
  )(adj, ans, mask)
  return sums.reshape(_B, nr), cnts.reshape(_B, nr)


@jax.jit
def kernel(adj, mask, gold_edge_index, gold_edge_attr, ent_indices):
  del ent_indices
  e = gold_edge_index.astype(jnp.int32)
  pos = e[:, :, 0] * _N + e[:, :, 1]
  attr = gold_edge_attr.astype(jnp.int32)
  ans = _sc_scatter(pos, attr)
  ans = ans.reshape(_B, _N, _N)
  sums, cnts = _tc_ce(adj, ans, mask)
  s = sums.sum(axis=1)
  c = cnts.sum(axis=1)
  return (s / jnp.maximum(c, 1.0)).sum() / _B


# speedup vs baseline: 5.6427x; 1.1330x over previous
import functools

import jax
import jax.numpy as jnp
from jax import lax
from jax.experimental import pallas as pl
from jax.experimental.pallas import tpu as pltpu
from jax.experimental.pallas import tpu_sc as plsc

_B, _C, _N, _E = 4, 16, 1024, 4096
_NSUB = 16
_NCORE = 2
_TWORDS = (_N * _N) // _NSUB
_TROWS = _N // _NSUB
_CHUNK = 16
_NCHUNKS = _E // _CHUNK
_BIG = 2 ** 30


def _sc_scatter_body(pos_hbm, attr_hbm, out_hbm, pos_v, attr_v, grid_v, key_v):
  cid = lax.axis_index("c")
  sid = lax.axis_index("s")
  lane = lax.iota(jnp.int32, _CHUNK)
  zeros16 = jnp.zeros((_CHUNK,), jnp.int32)

  for b in range(_B):
    @pl.when(cid == (b // 2))
    def _process():
      pltpu.sync_copy(pos_hbm.at[pl.ds(b * _E, _E)], pos_v)
      pltpu.sync_copy(attr_hbm.at[pl.ds(b * _E, _E)], attr_v)

      def _zero(i, _):
        row = lax.shift_right_logical(i, 4)
        base = jnp.bitwise_and(i, 15) * 64
        grid_v[row, pl.ds(base, 16)] = zeros16
        grid_v[row, pl.ds(base + 16, 16)] = zeros16
        grid_v[row, pl.ds(base + 32, 16)] = zeros16
        grid_v[row, pl.ds(base + 48, 16)] = zeros16
        return 0
      lax.fori_loop(0, _TWORDS // 64, _zero, 0)

      def _chunk(i, _):
        pv = pos_v[pl.ds(i * _CHUNK, _CHUNK)]
        av = attr_v[pl.ds(i * _CHUNK, _CHUNK)]
        own = lax.shift_right_logical(pv, 16) == sid
        lp = jnp.bitwise_and(pv, jnp.int32(_TWORDS - 1))
        key = jnp.where(own, lp * 16 + lane, jnp.int32(_BIG))
        sk, sa = plsc.sort_key_val(key, av)
        key_v[pl.ds(0, 16)] = sk
        nxt = key_v[pl.ds(1, 16)]
        grp = lax.shift_right_logical(sk, 4)
        ngrp = lax.shift_right_logical(nxt, 4)
        keep = jnp.logical_and(
            jnp.logical_or(grp != ngrp, lane == 15), sk < _BIG)
        a7 = jnp.bitwise_and(lax.shift_right_logical(grp, 7), 7)
        a10 = jnp.bitwise_and(lax.shift_right_logical(grp, 10), 7)
        toff = jnp.bitwise_or(
            jnp.bitwise_and(grp, jnp.int32(0xE07F)),
            jnp.bitwise_or(a7 * 1024, a10 * 128))
        plsc.store_scatter(
            grid_v,
            [lax.shift_right_logical(toff, 10), jnp.bitwise_and(toff, 1023)],
            sa, mask=keep)
        return 0
      lax.fori_loop(0, _NCHUNKS, _chunk, 0)

      pltpu.sync_copy(grid_v, out_hbm.at[b, pl.ds(sid * _TROWS, _TROWS)])


def _sc_scatter(pos, attr):
  mesh = plsc.VectorSubcoreMesh(core_axis_name="c", subcore_axis_name="s")
  return pl.kernel(
      _sc_scatter_body,
      out_type=jax.ShapeDtypeStruct((_B, _N, _N), jnp.int32),
      mesh=mesh,
      compiler_params=pltpu.CompilerParams(
          needs_layout_passes=False, use_tc_tiling_on_sc=True),
      scratch_types=[
          pltpu.VMEM((_E,), jnp.int32),
          pltpu.VMEM((_E,), jnp.int32),
          pltpu.VMEM((_TROWS, _N), jnp.int32),
          pltpu.VMEM((32,), jnp.int32),
      ],
  )(pos, attr)


_TR = 128


_SR = 8


def _tc_ce_body(*refs):
  adj_refs = refs[:_C]
  ans_ref, mask_ref, sum_ref, cnt_ref = refs[_C:]
  total = jnp.float32(0.0)
  count = jnp.float32(0.0)
  for t in range(_TR // _SR):
    r0 = t * _SR
    m = adj_refs[0][0, 0, r0:r0 + _SR, :]
    for c in range(1, _C):
      m = jnp.maximum(m, adj_refs[c][0, 0, r0:r0 + _SR, :])
    tgt = ans_ref[0, r0:r0 + _SR, :]
    s = jnp.zeros((_SR, _N), jnp.float32)
    sel = jnp.zeros((_SR, _N), jnp.float32)
    for c in range(_C):
      xc = adj_refs[c][0, 0, r0:r0 + _SR, :]
      s = s + jnp.exp(xc - m)
      sel = sel + jnp.where(tgt == c, xc, 0.0)
    mf = mask_ref[0, r0:r0 + _SR, :].astype(jnp.float32)
    nll = (jnp.log(s) + m - sel) * mf
    total = total + jnp.sum(nll)
    count = count + jnp.sum(mf)
  sum_ref[0, 0, 0] = total
  cnt_ref[0, 0, 0] = count


def _adj_spec(c):
  return pl.BlockSpec((1, 1, _TR, _N), lambda b, r, c=c: (b, c, r, 0))


def _tc_ce(adj, ans, mask):
  nr = _N // _TR
  sums, cnts = pl.pallas_call(
      _tc_ce_body,
      grid=(_B, nr),
      in_specs=[_adj_spec(c) for c in range(_C)] + [
          pl.BlockSpec((1, _TR, _N), lambda b, r: (b, r, 0)),
          pl.BlockSpec((1, _TR, _N), lambda b, r: (b, r, 0)),
      ],
      out_specs=[
          pl.BlockSpec((1, 1, 1), lambda b, r: (b * nr + r, 0, 0),
                       memory_space=pltpu.SMEM),
          pl.BlockSpec((1, 1, 1), lambda b, r: (b * nr + r, 0, 0),
                       memory_space=pltpu.SMEM),
      ],
      out_shape=[
          jax.ShapeDtypeStruct((_B * nr, 1, 1), jnp.float32),
          jax.ShapeDtypeStruct((_B * nr, 1, 1), jnp.float32),
      ],
  )(*([adj] * _C + [ans, mask]))
  return sums.reshape(_B, nr), cnts.reshape(_B, nr)


@jax.jit
def kernel(adj, mask, gold_edge_index, gold_edge_attr, ent_indices):
  del ent_indices
  e = gold_edge_index.astype(jnp.int32)
  pos = (e[:, :, 0] * _N + e[:, :, 1]).reshape(-1)
  attr = gold_edge_attr.astype(jnp.int32).reshape(-1)
  ans = _sc_scatter(pos, attr)
  sums, cnts = _tc_ce(adj, ans, mask)
  s = sums.sum(axis=1)
  c = cnts.sum(axis=1)
  return (s / jnp.maximum(c, 1.0)).sum() / _B


# speedup vs baseline: 5.9176x; 1.0487x over previous
import functools

import jax
import jax.numpy as jnp
from jax import lax
from jax.experimental import pallas as pl
from jax.experimental.pallas import tpu as pltpu
from jax.experimental.pallas import tpu_sc as plsc

_B, _C, _N, _E = 4, 16, 1024, 4096
_NSUB = 16
_NCORE = 2
_TWORDS = (_N * _N) // _NSUB
_TROWS = _N // _NSUB
_CHUNK = 16
_NCHUNKS = _E // _CHUNK
_BIG = 2 ** 30


def _sc_scatter_body(pos_hbm, attr_hbm, out_hbm, pos_v, attr_v, grid_v, key_v):
  cid = lax.axis_index("c")
  sid = lax.axis_index("s")
  lane = lax.iota(jnp.int32, _CHUNK)
  zeros16 = jnp.zeros((_CHUNK,), jnp.int32)

  for b in range(_B):
    @pl.when(cid == (b // 2))
    def _process():
      pltpu.sync_copy(pos_hbm.at[pl.ds(b * _E, _E)], pos_v)
      pltpu.sync_copy(attr_hbm.at[pl.ds(b * _E, _E)], attr_v)

      def _zero(i, _):
        row = lax.shift_right_logical(i, 4)
        base = jnp.bitwise_and(i, 15) * 64
        grid_v[row, pl.ds(base, 16)] = zeros16
        grid_v[row, pl.ds(base + 16, 16)] = zeros16
        grid_v[row, pl.ds(base + 32, 16)] = zeros16
        grid_v[row, pl.ds(base + 48, 16)] = zeros16
        return 0
      lax.fori_loop(0, _TWORDS // 64, _zero, 0)

      def _chunk(i, _):
        pv = pos_v[pl.ds(i * _CHUNK, _CHUNK)]
        av = attr_v[pl.ds(i * _CHUNK, _CHUNK)]
        own = lax.shift_right_logical(pv, 16) == sid
        lp = jnp.bitwise_and(pv, jnp.int32(_TWORDS - 1))
        key = jnp.where(own, lp * 16 + lane, jnp.int32(_BIG))
        sk, sa = plsc.sort_key_val(key, av)
        key_v[pl.ds(0, 16)] = sk
        nxt = key_v[pl.ds(1, 16)]
        grp = lax.shift_right_logical(sk, 4)
        ngrp = lax.shift_right_logical(nxt, 4)
        keep = jnp.logical_and(
            jnp.logical_or(grp != ngrp, lane == 15), sk < _BIG)
        a7 = jnp.bitwise_and(lax.shift_right_logical(grp, 7), 7)
        a10 = jnp.bitwise_and(lax.shift_right_logical(grp, 10), 7)
        toff = jnp.bitwise_or(
            jnp.bitwise_and(grp, jnp.int32(0xE07F)),
            jnp.bitwise_or(a7 * 1024, a10 * 128))
        plsc.store_scatter(
            grid_v,
            [lax.shift_right_logical(toff, 10), jnp.bitwise_and(toff, 1023)],
            sa, mask=keep)
        return 0
      lax.fori_loop(0, _NCHUNKS, _chunk, 0)

      pltpu.sync_copy(grid_v, out_hbm.at[b, pl.ds(sid * _TROWS, _TROWS)])


def _sc_scatter(pos, attr):
  mesh = plsc.VectorSubcoreMesh(core_axis_name="c", subcore_axis_name="s")
  return pl.kernel(
      _sc_scatter_body,
      out_type=jax.ShapeDtypeStruct((_B, _N, _N), jnp.int32),
      mesh=mesh,
      compiler_params=pltpu.CompilerParams(
          needs_layout_passes=False, use_tc_tiling_on_sc=True),
      scratch_types=[
          pltpu.VMEM((_E,), jnp.int32),
          pltpu.VMEM((_E,), jnp.int32),
          pltpu.VMEM((_TROWS, _N), jnp.int32),
          pltpu.VMEM((32,), jnp.int32),
      ],
  )(pos, attr)


_TR = 256


_SR = 8


def _tc_ce_body(*refs):
  adj_refs = refs[:_C]
  ans_ref, mask_ref, sum_ref, cnt_ref = refs[_C:]
  total = jnp.float32(0.0)
  count = jnp.float32(0.0)
  for t in range(_TR // _SR):
    r0 = t * _SR
    m = adj_refs[0][0, 0, r0:r0 + _SR, :]
    for c in range(1, _C):
      m = jnp.maximum(m, adj_refs[c][0, 0, r0:r0 + _SR, :])
    tgt = ans_ref[0, r0:r0 + _SR, :]
    s = jnp.zeros((_SR, _N), jnp.float32)
    sel = jnp.zeros((_SR, _N), jnp.float32)
    for c in range(_C):
      xc = adj_refs[c][0, 0, r0:r0 + _SR, :]
      s = s + jnp.exp(xc - m)
      sel = sel + jnp.where(tgt == c, xc, 0.0)
    mf = mask_ref[0, r0:r0 + _SR, :].astype(jnp.float32)
    nll = (jnp.log(s) + m - sel) * mf
    total = total + jnp.sum(nll)
    count = count + jnp.sum(mf)
  sum_ref[0, 0, 0] = total
  cnt_ref[0, 0, 0] = count


def _adj_spec(c):
  return pl.BlockSpec((1, 1, _TR, _N), lambda b, r, c=c: (b, c, r, 0))


def _tc_ce(adj, ans, mask):
  nr = _N // _TR
  sums, cnts = pl.pallas_call(
      _tc_ce_body,
      grid=(_B, nr),
      in_specs=[_adj_spec(c) for c in range(_C)] + [
          pl.BlockSpec((1, _TR, _N), lambda b, r: (b, r, 0)),
          pl.BlockSpec((1, _TR, _N), lambda b, r: (b, r, 0)),
      ],
      out_specs=[
          pl.BlockSpec((1, 1, 1), lambda b, r: (b * nr + r, 0, 0),
                       memory_space=pltpu.SMEM),
          pl.BlockSpec((1, 1, 1), lambda b, r: (b * nr + r, 0, 0),
                       memory_space=pltpu.SMEM),
      ],
      out_shape=[
          jax.ShapeDtypeStruct((_B * nr, 1, 1), jnp.float32),
          jax.ShapeDtypeStruct((_B * nr, 1, 1), jnp.float32),
      ],
  )(*([adj] * _C + [ans, mask]))
  return sums.reshape(_B, nr), cnts.reshape(_B, nr)


@jax.jit
def kernel(adj, mask, gold_edge_index, gold_edge_attr, ent_indices):
  del ent_indices
  e = gold_edge_index.astype(jnp.int32)
  pos = (e[:, :, 0] * _N + e[:, :, 1]).reshape(-1)
  attr = gold_edge_attr.astype(jnp.int32).reshape(-1)
  ans = _sc_scatter(pos, attr)
  sums, cnts = _tc_ce(adj, ans, mask)
  s = sums.sum(axis=1)
  c = cnts.sum(axis=1)
  return (s / jnp.maximum(c, 1.0)).sum() / _B


# speedup vs baseline: 5.9295x; 1.0020x over previous
import functools

import jax
import jax.numpy as jnp
from jax import lax
from jax.experimental import pallas as pl
from jax.experimental.pallas import tpu as pltpu
from jax.experimental.pallas import tpu_sc as plsc

_B, _C, _N, _E = 4, 16, 1024, 4096
_NSUB = 16
_NCORE = 2
_TWORDS = (_N * _N) // _NSUB
_TROWS = _N // _NSUB
_CHUNK = 16
_NCHUNKS = _E // _CHUNK
_BIG = 2 ** 30


def _sc_scatter_body(pos_hbm, attr_hbm, out_hbm, pos_v, attr_v, grid_v, key_v):
  cid = lax.axis_index("c")
  sid = lax.axis_index("s")
  lane = lax.iota(jnp.int32, _CHUNK)
  zeros16 = jnp.zeros((_CHUNK,), jnp.int32)

  for b in range(_B):
    @pl.when(cid == (b // 2))
    def _process():
      pltpu.sync_copy(pos_hbm.at[pl.ds(b * _E, _E)], pos_v)
      pltpu.sync_copy(attr_hbm.at[pl.ds(b * _E, _E)], attr_v)

      def _zero(i, _):
        row = lax.shift_right_logical(i, 4)
        base = jnp.bitwise_and(i, 15) * 64
        grid_v[row, pl.ds(base, 16)] = zeros16
        grid_v[row, pl.ds(base + 16, 16)] = zeros16
        grid_v[row, pl.ds(base + 32, 16)] = zeros16
        grid_v[row, pl.ds(base + 48, 16)] = zeros16
        return 0
      lax.fori_loop(0, _TWORDS // 64, _zero, 0)

      def _chunk(i, _):
        pv = pos_v[pl.ds(i * _CHUNK, _CHUNK)]
        av = attr_v[pl.ds(i * _CHUNK, _CHUNK)]
        own = lax.shift_right_logical(pv, 16) == sid
        lp = jnp.bitwise_and(pv, jnp.int32(_TWORDS - 1))
        key = jnp.where(own, lp * 16 + lane, jnp.int32(_BIG))
        sk, sa = plsc.sort_key_val(key, av)
        key_v[pl.ds(0, 16)] = sk
        nxt = key_v[pl.ds(1, 16)]
        grp = lax.shift_right_logical(sk, 4)
        ngrp = lax.shift_right_logical(nxt, 4)
        keep = jnp.logical_and(
            jnp.logical_or(grp != ngrp, lane == 15), sk < _BIG)
        a7 = jnp.bitwise_and(lax.shift_right_logical(grp, 7), 7)
        a10 = jnp.bitwise_and(lax.shift_right_logical(grp, 10), 7)
        toff = jnp.bitwise_or(
            jnp.bitwise_and(grp, jnp.int32(0xE07F)),
            jnp.bitwise_or(a7 * 1024, a10 * 128))
        plsc.store_scatter(
            grid_v,
            [lax.shift_right_logical(toff, 10), jnp.bitwise_and(toff, 1023)],
            sa, mask=keep)
        return 0
      lax.fori_loop(0, _NCHUNKS, _chunk, 0)

      pltpu.sync_copy(grid_v, out_hbm.at[b, pl.ds(sid * _TROWS, _TROWS)])


def _sc_scatter(pos, attr):
  mesh = plsc.VectorSubcoreMesh(core_axis_name="c", subcore_axis_name="s")
  return pl.kernel(
      _sc_scatter_body,
      out_type=jax.ShapeDtypeStruct((_B, _N, _N), jnp.int32),
      mesh=mesh,
      compiler_params=pltpu.CompilerParams(
          needs_layout_passes=False, use_tc_tiling_on_sc=True),
      scratch_types=[
          pltpu.VMEM((_E,), jnp.int32),
          pltpu.VMEM((_E,), jnp.int32),
          pltpu.VMEM((_TROWS, _N), jnp.int32),
          pltpu.VMEM((32,), jnp.int32),
      ],
  )(pos, attr)


_TR = 256


_SR = 8


_HR = _TR // 2


def _tc_ce_body(*refs):
  adj_refs = refs[:2 * _C]
  ans_ref, mask_ref, sum_ref, cnt_ref = refs[2 * _C:]
  total = jnp.float32(0.0)
  count = jnp.float32(0.0)
  for t in range(_TR // _SR):
    r0 = t * _SR
    h, rh = divmod(r0, _HR)
    m = adj_refs[h][0, 0, rh:rh + _SR, :]
    for c in range(1, _C):
      m = jnp.maximum(m, adj_refs[2 * c + h][0, 0, rh:rh + _SR, :])
    tgt = ans_ref[0, r0:r0 + _SR, :]
    s = jnp.zeros((_SR, _N), jnp.float32)
    sel = jnp.zeros((_SR, _N), jnp.float32)
    for c in range(_C):
      xc = adj_refs[2 * c + h][0, 0, rh:rh + _SR, :]
      s = s + jnp.exp(xc - m)
      sel = sel + jnp.where(tgt == c, xc, 0.0)
    mf = mask_ref[0, r0:r0 + _SR, :].astype(jnp.float32)
    nll = (jnp.log(s) + m - sel) * mf
    total = total + jnp.sum(nll)
    count = count + jnp.sum(mf)
  sum_ref[0, 0, 0] = total
  cnt_ref[0, 0, 0] = count


def _adj_spec(c, h):
  return pl.BlockSpec(
      (1, 1, _HR, _N), lambda b, r, c=c, h=h: (b, c, 2 * r + h, 0))


def _tc_ce(adj, ans, mask):
  nr = _N // _TR
  sums, cnts = pl.pallas_call(
      _tc_ce_body,
      grid=(_B, nr),
      in_specs=[_adj_spec(c, h) for c in range(_C) for h in range(2)] + [
          pl.BlockSpec((1, _TR, _N), lambda b, r: (b, r, 0)),
          pl.BlockSpec((1, _TR, _N), lambda b, r: (b, r, 0)),
      ],
      out_specs=[
          pl.BlockSpec((1, 1, 1), lambda b, r: (b * nr + r, 0, 0),
                       memory_space=pltpu.SMEM),
          pl.BlockSpec((1, 1, 1), lambda b, r: (b * nr + r, 0, 0),
                       memory_space=pltpu.SMEM),
      ],
      out_shape=[
          jax.ShapeDtypeStruct((_B * nr, 1, 1), jnp.float32),
          jax.ShapeDtypeStruct((_B * nr, 1, 1), jnp.float32),
      ],
  )(*([adj] * (2 * _C) + [ans, mask]))
  return sums.reshape(_B, nr), cnts.reshape(_B, nr)


@jax.jit
def kernel(adj, mask, gold_edge_index, gold_edge_attr, ent_indices):
  del ent_indices
  e = gold_edge_index.astype(jnp.int32)
  pos = (e[:, :, 0] * _N + e[:, :, 1]).reshape(-1)
  attr = gold_edge_attr.astype(jnp.int32).reshape(-1)
  ans = _sc_scatter(pos, attr)
  sums, cnts = _tc_ce(adj, ans, mask)
  s = sums.sum(axis=1)
  c = cnts.sum(axis=1)
  return (s / jnp.maximum(c, 1.0)).sum() / _B
